# baseline (speedup 1.0000x reference)
import numpy as np
import jax
import jax.numpy as jnp
from jax.experimental import pallas as pl
from jax.experimental.pallas import tpu as pltpu

_H = 224
_H1 = _H - 2
_POOL1 = 512
_H2 = (_POOL1 - 3) // 2 + 1
_P = 16
_D = _P * _P
_CO = 8
_VMEM_LIMIT = 48 * 1024 * 1024


def _pool_matrix(in_size, out_size):
    P = np.zeros((out_size, in_size), np.float32)
    for i in range(out_size):
        s = (i * in_size) // out_size
        e = -(-((i + 1) * in_size) // out_size)
        P[i, s:e] = 1.0 / (e - s)
    return P


def _build_operators():
    P1 = _pool_matrix(_H1, _POOL1)
    P2 = _pool_matrix(_H2, _P)
    Ls = []
    for d in range(3):
        R = np.zeros((_H2, _POOL1), np.float32)
        R[np.arange(_H2), 2 * np.arange(_H2) + d] = 1.0
        Ls.append(P2 @ R @ P1)
    L_all = np.concatenate(Ls, axis=0)
    emb = np.zeros((3, 48, _H), np.float32)
    for a in range(3):
        emb[a, :, a:a + _H1] = L_all
    Lrow = emb.reshape(144, _H)
    Lcolt = np.ascontiguousarray(np.transpose(emb, (0, 2, 1)))
    return Lrow, Lcolt


_LROW, _LCOLT = _build_operators()


def _make_fused_body(nb):
    def _fused_body(w1_ref, x_ref, lrow_ref, lcolt_ref, k2b_ref, o_ref):
        lrow = lrow_ref[...]
        for m in range(nb):
            A = [jnp.dot(lrow, x_ref[m, cp].astype(jnp.bfloat16),
                         preferred_element_type=jnp.float32)
                 for cp in range(3)]
            ustk = None
            for b in range(3):
                bcs = []
                for c in range(3):
                    bacc = None
                    for cp in range(3):
                        for a in range(3):
                            w = w1_ref[((c * 3 + cp) * 3 + a) * 3 + b]
                            t = w * A[cp][48 * a:48 * a + 48, :]
                            bacc = t if bacc is None else bacc + t
                    bcs.append(bacc)
                bcat = jnp.concatenate(bcs, axis=0)
                v = jnp.dot(bcat.astype(jnp.bfloat16), lcolt_ref[b],
                            preferred_element_type=jnp.float32)
                ustk = v if ustk is None else ustk + v
            res = jnp.dot(k2b_ref[...], ustk.astype(jnp.bfloat16),
                          preferred_element_type=jnp.float32)
            for o in range(_CO):
                s = 3 * o * _P
                z = (res[s:s + _P, 0:_P]
                     + res[s + _P:s + 2 * _P, _P:2 * _P]
                     + res[s + 2 * _P:s + 3 * _P, 2 * _P:3 * _P])
                o_ref[m, o] = z
    return _fused_body


def _make_dense_body(bm):
    def _dense_body(z_ref, wdr_ref, be_ref, o_ref):
        acc = None
        for i in range(_P):
            zi = z_ref[:, :, i, :].reshape(bm * _CO, _P)
            t = jnp.dot(zi, wdr_ref[i], preferred_element_type=jnp.float32)
            acc = t if acc is None else acc + t
        o_ref[...] = acc.reshape(bm, _CO, _D) + be_ref[...]
    return _dense_body


def kernel(x, conv1_w, conv1_b, conv2_w, conv2_b, dense_w, dense_b):
    N = x.shape[0]
    nb = 4 if N % 4 == 0 else 1
    lrow = jnp.asarray(_LROW).astype(jnp.bfloat16)
    lcolt = jnp.asarray(_LCOLT).astype(jnp.bfloat16)

    w1_flat = conv1_w.astype(jnp.float32).reshape(-1)
    k2 = conv2_w.astype(jnp.float32)

    eye = jnp.eye(_P, dtype=jnp.float32)
    k2big = jnp.einsum('ochw,ij->owichj', k2,
                       eye).reshape(24 * _P, 9 * _P).astype(jnp.bfloat16)

    wd = dense_w.astype(jnp.float32)
    zb = (conv2_b.astype(jnp.float32)
          + jnp.einsum('ochw,c->o', k2, conv1_b.astype(jnp.float32)))
    bias_eff = (dense_b.astype(jnp.float32)[None, :]
                + zb[:, None] * jnp.sum(wd, axis=1)[None, :])

    z4 = pl.pallas_call(
        _make_fused_body(nb),
        grid=(N // nb,),
        in_specs=[
            pl.BlockSpec(memory_space=pltpu.MemorySpace.SMEM),
            pl.BlockSpec((nb, 3, _H, _H), lambda n: (n, 0, 0, 0)),
            pl.BlockSpec((144, _H), lambda n: (0, 0)),
            pl.BlockSpec((3, _H, 48), lambda n: (0, 0, 0)),
            pl.BlockSpec((24 * _P, 9 * _P), lambda n: (0, 0)),
        ],
        out_specs=pl.BlockSpec((nb, _CO, _P, _P), lambda n: (n, 0, 0, 0)),
        out_shape=jax.ShapeDtypeStruct((N, _CO, _P, _P), jnp.float32),
        compiler_params=pltpu.CompilerParams(
            dimension_semantics=("parallel",),
            vmem_limit_bytes=_VMEM_LIMIT),
    )(w1_flat, x.astype(jnp.float32), lrow, lcolt, k2big)

    wdr = jnp.transpose(wd.reshape(_D, _P, _P), (1, 2, 0))
    bm = N // 2 if N % 2 == 0 else N
    out = pl.pallas_call(
        _make_dense_body(bm),
        grid=(N // bm,),
        in_specs=[
            pl.BlockSpec((bm, _CO, _P, _P), lambda i: (i, 0, 0, 0)),
            pl.BlockSpec((_P, _P, _D), lambda i: (0, 0, 0)),
            pl.BlockSpec((_CO, _D), lambda i: (0, 0)),
        ],
        out_specs=pl.BlockSpec((bm, _CO, _D), lambda i: (i, 0, 0)),
        out_shape=jax.ShapeDtypeStruct((N, _CO, _D), jnp.float32),
        compiler_params=pltpu.CompilerParams(
            dimension_semantics=("parallel",),
            vmem_limit_bytes=_VMEM_LIMIT),
    )(z4, wdr, bias_eff)
    return out


# speedup vs baseline: 1.0621x; 1.0621x over previous
import numpy as np
import jax
import jax.numpy as jnp
from jax.experimental import pallas as pl
from jax.experimental.pallas import tpu as pltpu

_H = 224
_H1 = _H - 2
_POOL1 = 512
_H2 = (_POOL1 - 3) // 2 + 1
_P = 16
_D = _P * _P
_CO = 8
_VMEM_LIMIT = 48 * 1024 * 1024


def _pool_matrix(in_size, out_size):
    P = np.zeros((out_size, in_size), np.float32)
    for i in range(out_size):
        s = (i * in_size) // out_size
        e = -(-((i + 1) * in_size) // out_size)
        P[i, s:e] = 1.0 / (e - s)
    return P


def _build_operators():
    P1 = _pool_matrix(_H1, _POOL1)
    P2 = _pool_matrix(_H2, _P)
    Ls = []
    for d in range(3):
        R = np.zeros((_H2, _POOL1), np.float32)
        R[np.arange(_H2), 2 * np.arange(_H2) + d] = 1.0
        Ls.append(P2 @ R @ P1)
    L_all = np.concatenate(Ls, axis=0)
    emb = np.zeros((3, 48, _H), np.float32)
    for a in range(3):
        emb[a, :, a:a + _H1] = L_all
    Lrow = emb.reshape(144, _H)
    Lcolt = np.ascontiguousarray(np.transpose(emb, (0, 2, 1)))
    return Lrow, Lcolt


_LROW, _LCOLT = _build_operators()


def _make_fused_body(nb):
    def _fused_body(w1_ref, x_ref, lrow_ref, lcolt_ref, k2b_ref, o_ref):
        lrow = lrow_ref[...]
        for m in range(nb):
            A = [jnp.dot(lrow, x_ref[m, cp].astype(jnp.bfloat16),
                         preferred_element_type=jnp.float32)
                 for cp in range(3)]
            res = None
            for c in range(3):
                U = None
                for b in range(3):
                    bacc = None
                    for cp in range(3):
                        for a in range(3):
                            w = w1_ref[((c * 3 + cp) * 3 + a) * 3 + b]
                            t = w * A[cp][48 * a:48 * a + 48, :]
                            bacc = t if bacc is None else bacc + t
                    ub = jnp.dot(bacc, lcolt_ref[b],
                                 preferred_element_type=jnp.float32)
                    U = ub if U is None else U + ub
                r = jnp.dot(k2b_ref[c], U,
                            preferred_element_type=jnp.float32)
                res = r if res is None else res + r
            for o in range(_CO):
                s = 3 * o * _P
                z = (res[s:s + _P, 0:_P]
                     + res[s + _P:s + 2 * _P, _P:2 * _P]
                     + res[s + 2 * _P:s + 3 * _P, 2 * _P:3 * _P])
                o_ref[m, o] = z
    return _fused_body


def _make_dense_body(bm):
    def _dense_body(z_ref, wdr_ref, be_ref, o_ref):
        acc = None
        for i in range(_P):
            zi = z_ref[:, :, i, :].reshape(bm * _CO, _P)
            t = jnp.dot(zi, wdr_ref[i], preferred_element_type=jnp.float32)
            acc = t if acc is None else acc + t
        o_ref[...] = acc.reshape(bm, _CO, _D) + be_ref[...]
    return _dense_body


def kernel(x, conv1_w, conv1_b, conv2_w, conv2_b, dense_w, dense_b):
    N = x.shape[0]
    nb = 4 if N % 4 == 0 else 1
    lrow = jnp.asarray(_LROW).astype(jnp.bfloat16)
    lcolt = jnp.asarray(_LCOLT)

    w1_flat = conv1_w.astype(jnp.float32).reshape(-1)
    k2 = conv2_w.astype(jnp.float32)

    eye = jnp.eye(_P, dtype=jnp.float32)
    k2big = jnp.einsum('ochw,ij->cowihj', k2,
                       eye).reshape(3, 24 * _P, 3 * _P)

    wd = dense_w.astype(jnp.float32)
    zb = (conv2_b.astype(jnp.float32)
          + jnp.einsum('ochw,c->o', k2, conv1_b.astype(jnp.float32)))
    bias_eff = (dense_b.astype(jnp.float32)[None, :]
                + zb[:, None] * jnp.sum(wd, axis=1)[None, :])

    z4 = pl.pallas_call(
        _make_fused_body(nb),
        grid=(N // nb,),
        in_specs=[
            pl.BlockSpec(memory_space=pltpu.MemorySpace.SMEM),
            pl.BlockSpec((nb, 3, _H, _H), lambda n: (n, 0, 0, 0)),
            pl.BlockSpec((144, _H), lambda n: (0, 0)),
            pl.BlockSpec((3, _H, 48), lambda n: (0, 0, 0)),
            pl.BlockSpec((3, 24 * _P, 3 * _P), lambda n: (0, 0, 0)),
        ],
        out_specs=pl.BlockSpec((nb, _CO, _P, _P), lambda n: (n, 0, 0, 0)),
        out_shape=jax.ShapeDtypeStruct((N, _CO, _P, _P), jnp.float32),
        compiler_params=pltpu.CompilerParams(
            dimension_semantics=("parallel",),
            vmem_limit_bytes=_VMEM_LIMIT),
    )(w1_flat, x.astype(jnp.float32), lrow, lcolt, k2big)

    wdr = jnp.transpose(wd.reshape(_D, _P, _P), (1, 2, 0))
    bm = N // 2 if N % 2 == 0 else N
    out = pl.pallas_call(
        _make_dense_body(bm),
        grid=(N // bm,),
        in_specs=[
            pl.BlockSpec((bm, _CO, _P, _P), lambda i: (i, 0, 0, 0)),
            pl.BlockSpec((_P, _P, _D), lambda i: (0, 0, 0)),
            pl.BlockSpec((_CO, _D), lambda i: (0, 0)),
        ],
        out_specs=pl.BlockSpec((bm, _CO, _D), lambda i: (i, 0, 0)),
        out_shape=jax.ShapeDtypeStruct((N, _CO, _D), jnp.float32),
        compiler_params=pltpu.CompilerParams(
            dimension_semantics=("parallel",),
            vmem_limit_bytes=_VMEM_LIMIT),
    )(z4, wdr, bias_eff)
    return out


# speedup vs baseline: 1.1163x; 1.0510x over previous
import numpy as np
import jax
import jax.numpy as jnp
from jax.experimental import pallas as pl
from jax.experimental.pallas import tpu as pltpu

_H = 224
_H1 = _H - 2
_POOL1 = 512
_H2 = (_POOL1 - 3) // 2 + 1
_P = 16
_D = _P * _P
_CO = 8
_VMEM_LIMIT = 48 * 1024 * 1024


def _pool_matrix(in_size, out_size):
    P = np.zeros((out_size, in_size), np.float32)
    for i in range(out_size):
        s = (i * in_size) // out_size
        e = -(-((i + 1) * in_size) // out_size)
        P[i, s:e] = 1.0 / (e - s)
    return P


def _build_operators():
    P1 = _pool_matrix(_H1, _POOL1)
    P2 = _pool_matrix(_H2, _P)
    Ls = []
    for d in range(3):
        R = np.zeros((_H2, _POOL1), np.float32)
        R[np.arange(_H2), 2 * np.arange(_H2) + d] = 1.0
        Ls.append(P2 @ R @ P1)
    L_all = np.concatenate(Ls, axis=0)
    emb = np.zeros((3, 48, _H), np.float32)
    for a in range(3):
        emb[a, :, a:a + _H1] = L_all
    Lrow = emb.reshape(144, _H)
    Lcolt = np.ascontiguousarray(np.transpose(emb, (0, 2, 1)))
    return Lrow, Lcolt


_LROW, _LCOLT = _build_operators()


def _make_fused_body(nb):
    def _fused_body(w1_ref, x_ref, lrow_ref, lcolt_ref, k2c_ref, o_ref):
        lrow = lrow_ref[...]
        for m in range(nb):
            A = [jnp.dot(lrow, x_ref[m, cp].astype(jnp.bfloat16),
                         preferred_element_type=jnp.float32)
                 for cp in range(3)]
            res = None
            for c in range(3):
                U = None
                for b in range(3):
                    bacc = None
                    for cp in range(3):
                        for a in range(3):
                            w = w1_ref[((c * 3 + cp) * 3 + a) * 3 + b]
                            t = w * A[cp][48 * a:48 * a + 48, :]
                            bacc = t if bacc is None else bacc + t
                    ub = jnp.dot(bacc, lcolt_ref[b],
                                 preferred_element_type=jnp.float32)
                    U = ub if U is None else U + ub
                r = jnp.dot(k2c_ref[c], U,
                            preferred_element_type=jnp.float32)
                res = r if res is None else res + r
            for o in range(_CO):
                s = 3 * o * _P
                z = (res[s:s + _P, 0:_P]
                     + res[s + _P:s + 2 * _P, _P:2 * _P]
                     + res[s + 2 * _P:s + 3 * _P, 2 * _P:3 * _P])
                o_ref[m, o] = z
    return _fused_body


def _make_dense_body(bm):
    def _dense_body(z_ref, wdr_ref, be_ref, o_ref):
        acc = None
        for i in range(_P):
            zi = z_ref[:, :, i, :].reshape(bm * _CO, _P)
            t = jnp.dot(zi, wdr_ref[i], preferred_element_type=jnp.float32)
            acc = t if acc is None else acc + t
        o_ref[...] = acc.reshape(bm, _CO, _D) + be_ref[...]
    return _dense_body


def kernel(x, conv1_w, conv1_b, conv2_w, conv2_b, dense_w, dense_b):
    N = x.shape[0]
    nb = 8 if N % 8 == 0 else 1
    lrow = jnp.asarray(_LROW).astype(jnp.bfloat16)
    lcolt = jnp.asarray(_LCOLT)

    w1_flat = conv1_w.astype(jnp.float32).reshape(-1)
    k2 = conv2_w.astype(jnp.float32)

    eye = jnp.eye(_P, dtype=jnp.float32)
    k2c = jnp.einsum('ochw,ij->cowihj', k2,
                     eye).reshape(3, 24 * _P, 3 * _P)

    wd = dense_w.astype(jnp.float32)
    zb = (conv2_b.astype(jnp.float32)
          + jnp.einsum('ochw,c->o', k2, conv1_b.astype(jnp.float32)))
    bias_eff = (dense_b.astype(jnp.float32)[None, :]
                + zb[:, None] * jnp.sum(wd, axis=1)[None, :])

    z4 = pl.pallas_call(
        _make_fused_body(nb),
        grid=(N // nb,),
        in_specs=[
            pl.BlockSpec(memory_space=pltpu.MemorySpace.SMEM),
            pl.BlockSpec((nb, 3, _H, _H), lambda n: (n, 0, 0, 0)),
            pl.BlockSpec((144, _H), lambda n: (0, 0)),
            pl.BlockSpec((3, _H, 48), lambda n: (0, 0, 0)),
            pl.BlockSpec((3, 24 * _P, 3 * _P), lambda n: (0, 0, 0)),
        ],
        out_specs=pl.BlockSpec((nb, _CO, _P, _P), lambda n: (n, 0, 0, 0)),
        out_shape=jax.ShapeDtypeStruct((N, _CO, _P, _P), jnp.float32),
        compiler_params=pltpu.CompilerParams(
            dimension_semantics=("parallel",),
            vmem_limit_bytes=_VMEM_LIMIT),
    )(w1_flat, x.astype(jnp.float32), lrow, lcolt, k2c)

    wdr = jnp.transpose(wd.reshape(_D, _P, _P), (1, 2, 0))
    bm = N // 2 if N % 2 == 0 else N
    out = pl.pallas_call(
        _make_dense_body(bm),
        grid=(N // bm,),
        in_specs=[
            pl.BlockSpec((bm, _CO, _P, _P), lambda i: (i, 0, 0, 0)),
            pl.BlockSpec((_P, _P, _D), lambda i: (0, 0, 0)),
            pl.BlockSpec((_CO, _D), lambda i: (0, 0)),
        ],
        out_specs=pl.BlockSpec((bm, _CO, _D), lambda i: (i, 0, 0)),
        out_shape=jax.ShapeDtypeStruct((N, _CO, _D), jnp.float32),
        compiler_params=pltpu.CompilerParams(
            dimension_semantics=("parallel",),
            vmem_limit_bytes=_VMEM_LIMIT),
    )(z4, wdr, bias_eff)
    return out
